# baseline (speedup 1.0000x reference)
import functools

import jax
import jax.numpy as jnp
from jax.experimental import pallas as pl
from jax.experimental.pallas import tpu as pltpu


def _prep_body(g_ref, r_ref, a16_ref, *, bi):
    i = pl.program_id(0)
    a = g_ref[...]
    row = jax.lax.broadcasted_iota(jnp.int32, a.shape, 0) + i * bi
    col = jax.lax.broadcasted_iota(jnp.int32, a.shape, 1)
    a = jnp.where(row == col, 1.0, a)
    a16_ref[...] = a.astype(jnp.bfloat16)
    r_ref[...] = jnp.sum(a, axis=1, keepdims=True)


def _mega_body(a16_ref, r_ref, ft_ref, w0_ref, w1_ref, w2_ref,
               b0_ref, b1_ref, b2_ref, out_ref,
               s_ref, u_ref, acc_ref, za_ref, zb_ref, *, bi, ni):
    l = pl.program_id(0)
    i = pl.program_id(1)
    isl = pl.ds(i * bi, bi)
    a16 = a16_ref[isl, :]

    @pl.when(l == 0)
    def _():
        p = jax.lax.rsqrt(r_ref[:, isl])
        part = jnp.dot(p, a16.astype(jnp.float32),
                       preferred_element_type=jnp.float32)

        @pl.when(i == 0)
        def _():
            s_ref[...] = part

        @pl.when(i != 0)
        def _():
            s_ref[...] = s_ref[...] + part

        @pl.when(i == ni - 1)
        def _():
            u_ref[...] = jax.lax.rsqrt(jnp.sqrt(r_ref[...]) * s_ref[...])

    def layer(zsrc, wt_ref, b_ref, writeback):
        fout = wt_ref.shape[0]
        y = jnp.dot(wt_ref[...], zsrc.astype(jnp.float32),
                    preferred_element_type=jnp.float32)
        y = (y * u_ref[:, isl]).astype(jnp.bfloat16)
        part = jnp.dot(y, a16, preferred_element_type=jnp.float32)

        @pl.when(i == 0)
        def _():
            acc_ref[:fout, :] = part

        @pl.when(i != 0)
        def _():
            acc_ref[:fout, :] = acc_ref[:fout, :] + part

        @pl.when(i == ni - 1)
        def _():
            writeback(jnp.maximum(
                acc_ref[:fout, :] * u_ref[...] + b_ref[...], 0.0))

    @pl.when(l == 1)
    def _():
        layer(ft_ref[...], w0_ref, b0_ref,
              lambda v: za_ref.__setitem__((Ellipsis,), v.astype(jnp.bfloat16)))

    @pl.when(l == 2)
    def _():
        layer(za_ref[:, isl], w1_ref, b1_ref,
              lambda v: zb_ref.__setitem__((Ellipsis,), v.astype(jnp.bfloat16)))

    @pl.when(l == 3)
    def _():
        layer(zb_ref[:, isl], w2_ref, b2_ref,
              lambda v: out_ref.__setitem__((Ellipsis,), v))


def kernel(features, graph, W0, b0, W1, b1, W2, b2):
    n = graph.shape[0]
    bi = 512
    ni = n // bi
    d0 = W0.shape[0]
    h = W0.shape[1]
    latent = W2.shape[1]

    r, a16 = pl.pallas_call(
        functools.partial(_prep_body, bi=bi),
        grid=(ni,),
        in_specs=[pl.BlockSpec((bi, n), lambda i: (i, 0))],
        out_specs=[
            pl.BlockSpec((bi, 1), lambda i: (i, 0)),
            pl.BlockSpec((bi, n), lambda i: (i, 0)),
        ],
        out_shape=[
            jax.ShapeDtypeStruct((n, 1), jnp.float32),
            jax.ShapeDtypeStruct((n, n), jnp.bfloat16),
        ],
        compiler_params=pltpu.CompilerParams(
            dimension_semantics=("arbitrary",)
        ),
    )(graph)

    full = lambda shape: pl.BlockSpec(shape, lambda l, i: (0, 0))
    outt = pl.pallas_call(
        functools.partial(_mega_body, bi=bi, ni=ni),
        grid=(4, ni),
        in_specs=[
            full((n, n)),
            full((1, n)),
            pl.BlockSpec((d0, bi), lambda l, i: (0, jnp.where(l == 1, i, 0))),
            full((h, d0)),
            full((h, h)),
            full((latent, h)),
            full((h, 1)),
            full((h, 1)),
            full((latent, 1)),
        ],
        out_specs=full((latent, n)),
        out_shape=jax.ShapeDtypeStruct((latent, n), jnp.float32),
        scratch_shapes=[
            pltpu.VMEM((1, n), jnp.float32),
            pltpu.VMEM((1, n), jnp.float32),
            pltpu.VMEM((h, n), jnp.float32),
            pltpu.VMEM((h, n), jnp.bfloat16),
            pltpu.VMEM((h, n), jnp.bfloat16),
        ],
        compiler_params=pltpu.CompilerParams(
            dimension_semantics=("arbitrary", "arbitrary")
        ),
    )(a16, r.reshape(1, n), features.T, W0.T, W1.T, W2.T,
      b0.reshape(h, 1), b1.reshape(h, 1), b2.reshape(latent, 1))
    return outt.T
